# baseline (speedup 1.0000x reference)
import functools

import jax
import jax.numpy as jnp
from jax import lax
from jax.experimental import pallas as pl
from jax.experimental.pallas import tpu as pltpu
from jax.experimental.pallas import tpu_sc as plsc

VOCAB = 100000
EMBED_DIM = 128
BATCH = 4096
HIST = 200

N = BATCH * HIST
NC, NS = 2, 16
NW = NC * NS
N_SC = 184 * 128 * NW
B_PER_W = N_SC // NW
IW = 128
NIV = B_PER_W // IW
R = IW
NG = B_PER_W // R
NBUF = 4


def _make_kernel():
  mesh = plsc.VectorSubcoreMesh(core_axis_name="c", subcore_axis_name="s")

  @functools.partial(
      pl.kernel,
      mesh=mesh,
      out_type=jax.ShapeDtypeStruct((N_SC, EMBED_DIM), jnp.float32),
      scratch_types=[
          pltpu.VMEM((NIV, IW), jnp.int32),
          pltpu.VMEM((NBUF, R, EMBED_DIM), jnp.float32),
          [pltpu.SemaphoreType.DMA] * NBUF,
          [pltpu.SemaphoreType.DMA] * NBUF,
      ],
  )
  def k(x_hbm, table_hbm, out_hbm, idx_v, rows_v, gsem, wsem):
    wid = lax.axis_index("s") * NC + lax.axis_index("c")
    base = wid * B_PER_W

    pltpu.sync_copy(x_hbm.at[pl.ds(wid * NIV, NIV)], idx_v.at[...])

    def gather(g, b):
      pltpu.async_copy(table_hbm.at[idx_v.at[g]], rows_v.at[b], gsem[b])

    def gather_wait(g, b):
      pltpu.make_async_copy(
          table_hbm.at[idx_v.at[g]], rows_v.at[b], gsem[b]).wait()

    def write(g, b):
      pltpu.async_copy(
          rows_v.at[b], out_hbm.at[pl.ds(base + g * R, R)], wsem[b])

    def write_wait(g, b):
      pltpu.make_async_copy(
          rows_v.at[b], out_hbm.at[pl.ds(base + g * R, R)], wsem[b]).wait()

    for g in range(NBUF - 1):
      gather(g, g)

    def body(g):
      for b in range(NBUF):
        gg = g + b
        pb = (b + NBUF - 1) % NBUF
        @pl.when(gg >= 1)
        def _():
          write_wait(gg - 1, pb)
        @pl.when(gg + NBUF - 1 < NG)
        def _():
          gather(gg + NBUF - 1, pb)
        gather_wait(gg, b)
        write(gg, b)

    pl.loop(0, NG, step=NBUF)(body)

    write_wait(NG - 1, (NG - 1) % NBUF)

  return k


_gather_kernel = _make_kernel()


@jax.jit
def kernel(x, table):
  flat = x.reshape(N).astype(jnp.int32)
  out_sc = _gather_kernel(flat[:N_SC].reshape(N_SC // IW, IW), table)
  out_tc = jnp.take(table, flat[N_SC:], axis=0)
  return jnp.concatenate([out_sc, out_tc], axis=0).reshape(
      BATCH, HIST, EMBED_DIM)


# speedup vs baseline: 1.8784x; 1.8784x over previous
import functools

import jax
import jax.numpy as jnp
from jax import lax
from jax.experimental import pallas as pl
from jax.experimental.pallas import tpu as pltpu
from jax.experimental.pallas import tpu_sc as plsc

VOCAB = 100000
EMBED_DIM = 128
BATCH = 4096
HIST = 200

N = BATCH * HIST
NC, NS = 2, 16
NW = NC * NS
B_PER_W = N // NW
IW = 128
NIV = B_PER_W // IW
R = IW
NG = B_PER_W // R
NBUF = 4


def _make_kernel():
  mesh = plsc.VectorSubcoreMesh(core_axis_name="c", subcore_axis_name="s")

  @functools.partial(
      pl.kernel,
      mesh=mesh,
      out_type=jax.ShapeDtypeStruct((N, EMBED_DIM), jnp.float32),
      scratch_types=[
          pltpu.VMEM((NIV, IW), jnp.int32),
          pltpu.VMEM((NBUF, R, EMBED_DIM), jnp.float32),
          [pltpu.SemaphoreType.DMA] * NBUF,
          [pltpu.SemaphoreType.DMA] * NBUF,
      ],
  )
  def k(x_hbm, table_hbm, out_hbm, idx_v, rows_v, gsem, wsem):
    wid = lax.axis_index("s") * NC + lax.axis_index("c")
    base = wid * B_PER_W

    pltpu.sync_copy(x_hbm.at[pl.ds(wid * NIV, NIV)], idx_v.at[...])

    def gather(g, b):
      pltpu.async_copy(table_hbm.at[idx_v.at[g]], rows_v.at[b], gsem[b])

    def gather_wait(g, b):
      pltpu.make_async_copy(
          table_hbm.at[idx_v.at[g]], rows_v.at[b], gsem[b]).wait()

    def write(g, b):
      pltpu.async_copy(
          rows_v.at[b], out_hbm.at[pl.ds(base + g * R, R)], wsem[b])

    def write_wait(g, b):
      pltpu.make_async_copy(
          rows_v.at[b], out_hbm.at[pl.ds(base + g * R, R)], wsem[b]).wait()

    for g in range(NBUF - 1):
      gather(g, g)

    def body(g):
      for b in range(NBUF):
        gg = g + b
        pb = (b + NBUF - 1) % NBUF
        @pl.when(gg >= 1)
        def _():
          write_wait(gg - 1, pb)
        @pl.when(gg + NBUF - 1 < NG)
        def _():
          gather(gg + NBUF - 1, pb)
        gather_wait(gg, b)
        write(gg, b)

    pl.loop(0, NG, step=NBUF)(body)

    write_wait(NG - 1, (NG - 1) % NBUF)

  return k


_gather_kernel = _make_kernel()


@jax.jit
def kernel(x, table):
  flat = x.reshape(N // IW, IW).astype(jnp.int32)
  out = _gather_kernel(flat, table)
  return out.reshape(BATCH, HIST, EMBED_DIM)
